# baseline (speedup 1.0000x reference)
import functools

import jax
import jax.numpy as jnp
from jax import lax
from jax.experimental import pallas as pl
from jax.experimental.pallas import tpu as pltpu
from jax.experimental.pallas import tpu_sc as plsc

K = 2
CF = 1.25
BAL_COEF = 0.01
Z_COEF = 0.001

try:
    _SC_INFO = plsc.get_sparse_core_info()
    _NC = _SC_INFO.num_cores
    _NS = _SC_INFO.num_subcores
    _LN = _SC_INFO.num_lanes
except Exception:
    _NC, _NS, _LN = 2, 16, 16
_NW = _NC * _NS


def _router_body(C, B, N, E, x_ref, wg_ref,
                 slotw_ref, slotr_ref, gate_ref, aux_ref,
                 runcnt_ref, imp_ref, kept_ref, z_ref):
    k = pl.program_id(0)
    b = pl.program_id(1)
    nb = pl.num_programs(1)

    @pl.when(jnp.logical_and(k == 0, b == 0))
    def _init():
        runcnt_ref[...] = jnp.zeros_like(runcnt_ref)
        imp_ref[...] = jnp.zeros_like(imp_ref)
        kept_ref[...] = jnp.zeros_like(kept_ref)
        z_ref[...] = jnp.zeros_like(z_ref)

    xb = x_ref[...]
    logits = jnp.dot(xb, wg_ref[...], preferred_element_type=jnp.float32)
    mx = jnp.max(logits, axis=1, keepdims=True)
    ex = jnp.exp(logits - mx)
    se = jnp.sum(ex, axis=1, keepdims=True)
    probs = ex / se
    lse = mx[:, 0] + jnp.log(se[:, 0])

    iota_e = lax.broadcasted_iota(jnp.int32, (B, E), 1)
    m1 = jnp.max(probs, axis=1, keepdims=True)
    i1 = jnp.min(jnp.where(probs == m1, iota_e, E), axis=1, keepdims=True)
    oh1 = iota_e == i1
    probs2 = jnp.where(oh1, -jnp.inf, probs)
    m2 = jnp.max(probs2, axis=1, keepdims=True)
    i2 = jnp.min(jnp.where(probs2 == m2, iota_e, E), axis=1, keepdims=True)

    is_k0 = k == 0
    ek = jnp.where(is_k0, i1, i2)
    ohk = iota_e == ek
    gk = jnp.where(is_k0, m1, m2)[:, 0] / (m1 + m2)[:, 0]

    ohf = ohk.astype(jnp.float32)
    r_i = lax.broadcasted_iota(jnp.int32, (B, B), 0)
    c_i = lax.broadcasted_iota(jnp.int32, (B, B), 1)
    tril = (r_i > c_i).astype(jnp.float32)
    prior = jnp.dot(tril, ohf, preferred_element_type=jnp.float32)
    pos = jnp.sum((runcnt_ref[...] + prior) * ohf, axis=1).astype(jnp.int32)
    runcnt_ref[...] = runcnt_ref[...] + jnp.sum(ohf, axis=0, keepdims=True)

    keep = pos < C
    e_flat = ek[:, 0]
    slotw_ref[...] = jnp.where(keep, e_flat * C + pos, E * C)[:, None]
    slotr_ref[...] = (e_flat * C + jnp.minimum(pos, C - 1))[:, None]
    gate_ref[...] = jnp.where(keep, gk, 0.0)[:, None]

    kept_ref[...] = kept_ref[...] + jnp.sum(
        jnp.where(keep[:, None], ohf, 0.0), axis=0, keepdims=True)

    @pl.when(is_k0)
    def _acc():
        imp_ref[...] = imp_ref[...] + jnp.sum(probs, axis=0, keepdims=True)
        z_ref[...] = z_ref[...] + jnp.sum(lse * lse).reshape(1, 1)

    @pl.when(jnp.logical_and(k == 1, b == nb - 1))
    def _final():
        imp = imp_ref[...] / N
        frac = kept_ref[...] / N
        bal = E * jnp.sum(frac * imp)
        z = z_ref[0, 0] / N
        aux_ref[...] = jnp.full((8, 128), BAL_COEF * bal + Z_COEF * z,
                                jnp.float32)


def _router(x, Wg, C, interpret=False):
    N, D = x.shape
    E = Wg.shape[1]
    B = 1024
    nb = N // B
    return pl.pallas_call(
        functools.partial(_router_body, C, B, N, E),
        grid=(2, nb),
        in_specs=[
            pl.BlockSpec((B, D), lambda k, b: (b, 0)),
            pl.BlockSpec((D, E), lambda k, b: (0, 0)),
        ],
        out_specs=[
            pl.BlockSpec((B, 1), lambda k, b, _nb=nb: (k * _nb + b, 0)),
            pl.BlockSpec((B, 1), lambda k, b, _nb=nb: (k * _nb + b, 0)),
            pl.BlockSpec((B, 1), lambda k, b, _nb=nb: (k * _nb + b, 0)),
            pl.BlockSpec((8, 128), lambda k, b: (0, 0)),
        ],
        out_shape=[
            jax.ShapeDtypeStruct((2 * N, 1), jnp.int32),
            jax.ShapeDtypeStruct((2 * N, 1), jnp.int32),
            jax.ShapeDtypeStruct((2 * N, 1), jnp.float32),
            jax.ShapeDtypeStruct((8, 128), jnp.float32),
        ],
        scratch_shapes=[
            pltpu.VMEM((1, E), jnp.float32),
            pltpu.VMEM((1, E), jnp.float32),
            pltpu.VMEM((1, E), jnp.float32),
            pltpu.VMEM((1, 1), jnp.float32),
        ],
        interpret=interpret,
    )(x, Wg)


def _dispatch(x, slotw, C, E):
    N, D = x.shape
    BD = 32
    steps = N // (_NW * BD)
    mesh = plsc.VectorSubcoreMesh(core_axis_name="c", subcore_axis_name="s")

    @functools.partial(
        pl.kernel,
        out_type=jax.ShapeDtypeStruct((E * C + 8, D), jnp.float32),
        mesh=mesh,
        compiler_params=pltpu.CompilerParams(needs_layout_passes=False),
        scratch_types=[
            pltpu.VMEM((BD, D), jnp.float32),
            pltpu.VMEM((BD, D), jnp.float32),
            pltpu.VMEM((BD,), jnp.int32),
            pltpu.VMEM((BD,), jnp.int32),
            pltpu.VMEM((BD,), jnp.int32),
            pltpu.VMEM((BD,), jnp.int32),
            pltpu.SemaphoreType.DMA,
            pltpu.SemaphoreType.DMA,
            pltpu.SemaphoreType.DMA,
            pltpu.SemaphoreType.DMA,
        ],
    )
    def _disp(x_hbm, slotw_hbm, buf_hbm, xv0, xv1, i0a, i0b, i1a, i1b,
              xsem0, xsem1, ssem0, ssem1):
        wid = lax.axis_index("s") * _NC + lax.axis_index("c")
        xv = (xv0, xv1)
        i0 = (i0a, i0b)
        i1 = (i1a, i1b)
        xsem = (xsem0, xsem1)
        ssem = (ssem0, ssem1)
        tok0 = wid * (steps * BD)

        def start_in(s, p):
            base = tok0 + s * BD
            return (
                pltpu.async_copy(x_hbm.at[pl.ds(base, BD)], xv[p], xsem[p]),
                pltpu.async_copy(slotw_hbm.at[pl.ds(base, BD)], i0[p],
                                 xsem[p]),
                pltpu.async_copy(slotw_hbm.at[pl.ds(N + base, BD)], i1[p],
                                 xsem[p]),
            )

        xcp = [start_in(0, 0), None]
        scat = [None, None]
        for s in range(steps):
            p = s & 1
            for cp in xcp[p]:
                cp.wait()
            scat[p] = (
                pltpu.async_copy(xv[p], buf_hbm.at[i0[p]], ssem[p]),
                pltpu.async_copy(xv[p], buf_hbm.at[i1[p]], ssem[p]),
            )
            if s >= 1:
                scat[1 - p][0].wait()
                scat[1 - p][1].wait()
            if s + 1 < steps:
                xcp[1 - p] = start_in(s + 1, 1 - p)
        scat[(steps - 1) & 1][0].wait()
        scat[(steps - 1) & 1][1].wait()

    return _disp(x, slotw)


def _ffn_body(buf_ref, w1_ref, b1_ref, w2_ref, b2_ref, y_ref):
    h = jnp.dot(buf_ref[...], w1_ref[0],
                preferred_element_type=jnp.float32) + b1_ref[0]
    h = jax.nn.gelu(h).astype(jnp.bfloat16)
    y_ref[...] = jnp.dot(h, w2_ref[0],
                         preferred_element_type=jnp.float32) + b2_ref[0]


def _ffn(buf, W1, b1, W2, b2, C, interpret=False):
    E, D, DFF = W1.shape
    BC = 512
    nc = C // BC
    return pl.pallas_call(
        _ffn_body,
        grid=(E, nc),
        in_specs=[
            pl.BlockSpec((BC, D), lambda e, c, _nc=nc: (e * _nc + c, 0)),
            pl.BlockSpec((1, D, DFF), lambda e, c: (e, 0, 0)),
            pl.BlockSpec((1, 1, DFF), lambda e, c: (e, 0, 0)),
            pl.BlockSpec((1, DFF, D), lambda e, c: (e, 0, 0)),
            pl.BlockSpec((1, 1, D), lambda e, c: (e, 0, 0)),
        ],
        out_specs=pl.BlockSpec((BC, D), lambda e, c, _nc=nc: (e * _nc + c, 0)),
        out_shape=jax.ShapeDtypeStruct((E * C, D), jnp.float32),
        interpret=interpret,
    )(buf, W1, b1.reshape(E, 1, DFF), W2, b2.reshape(E, 1, D))


def _combine(y, slotr, gate, N, D):
    BC = 16
    tok_pw = N // _NW
    steps = tok_pw // BC
    mesh = plsc.VectorSubcoreMesh(core_axis_name="c", subcore_axis_name="s")

    @functools.partial(
        pl.kernel,
        out_type=jax.ShapeDtypeStruct((N, D), jnp.float32),
        mesh=mesh,
        compiler_params=pltpu.CompilerParams(needs_layout_passes=False),
        scratch_types=[
            pltpu.VMEM((BC, D), jnp.float32),
            pltpu.VMEM((BC, D), jnp.float32),
            pltpu.VMEM((BC, D), jnp.float32),
            pltpu.VMEM((BC, D), jnp.float32),
            pltpu.VMEM((BC, D), jnp.float32),
            pltpu.VMEM((BC, D), jnp.float32),
            pltpu.VMEM((tok_pw,), jnp.int32),
            pltpu.VMEM((tok_pw,), jnp.int32),
            pltpu.VMEM((tok_pw,), jnp.float32),
            pltpu.VMEM((tok_pw,), jnp.float32),
            pltpu.SemaphoreType.DMA,
            pltpu.SemaphoreType.DMA,
            pltpu.SemaphoreType.DMA,
            pltpu.SemaphoreType.DMA,
        ],
    )
    def _comb(y_hbm, slotr_hbm, gate_hbm, out_hbm,
              y0a, y0b, y1a, y1b, ova, ovb, i0all, i1all, g0all, g1all,
              gsem0, gsem1, osem0, osem1):
        wid = lax.axis_index("s") * _NC + lax.axis_index("c")
        tok0 = wid * tok_pw
        y0 = (y0a, y0b)
        y1 = (y1a, y1b)
        ov = (ova, ovb)
        gsem = (gsem0, gsem1)
        osem = (osem0, osem1)

        pltpu.sync_copy(slotr_hbm.at[pl.ds(tok0, tok_pw)], i0all)
        pltpu.sync_copy(slotr_hbm.at[pl.ds(N + tok0, tok_pw)], i1all)
        pltpu.sync_copy(gate_hbm.at[pl.ds(tok0, tok_pw)], g0all)
        pltpu.sync_copy(gate_hbm.at[pl.ds(N + tok0, tok_pw)], g1all)

        def start_gathers(s, p):
            pltpu.async_copy(
                y_hbm.at[i0all.at[pl.ds(s * BC, BC)]], y0[p], gsem[p])
            pltpu.async_copy(
                y_hbm.at[i1all.at[pl.ds(s * BC, BC)]], y1[p], gsem[p])

        start_gathers(0, 0)
        start_gathers(1, 1)

        def outer(i, carry):
            for p in (0, 1):
                s = 2 * i + p
                pltpu.make_async_copy(
                    y_hbm.at[pl.ds(0, BC)], y0[p], gsem[p]).wait()
                pltpu.make_async_copy(
                    y_hbm.at[pl.ds(0, BC)], y1[p], gsem[p]).wait()

                @pl.when(s >= 2)
                def _wait_out(p=p):
                    pltpu.make_async_copy(
                        ov[p], out_hbm.at[pl.ds(tok0, BC)], osem[p]).wait()

                def tbody(t, c, p=p, s=s):
                    tt = jnp.zeros((_LN,), jnp.int32) + (s * BC + t)
                    g0 = plsc.load_gather(g0all, [tt])
                    g1 = plsc.load_gather(g1all, [tt])
                    for cc in range(D // _LN):
                        sl = pl.ds(cc * _LN, _LN)
                        ov[p][t, sl] = g0 * y0[p][t, sl] + g1 * y1[p][t, sl]
                    return c

                lax.fori_loop(0, BC, tbody, 0)
                pltpu.async_copy(
                    ov[p], out_hbm.at[pl.ds(tok0 + s * BC, BC)], osem[p])

                @pl.when(s + 2 < steps)
                def _next(p=p, s=s):
                    start_gathers(s + 2, p)
            return carry

        lax.fori_loop(0, steps // 2, outer, 0)
        for p in (0, 1):
            pltpu.make_async_copy(
                ov[p], out_hbm.at[pl.ds(tok0, BC)], osem[p]).wait()

    return _comb(y, slotr, gate)


def kernel(x, Wg, W1, b1, W2, b2):
    N, D = x.shape
    E = Wg.shape[1]
    C = int(CF * N * K / E)

    slotw, slotr, gate, aux = _router(x, Wg, C)
    slotw = slotw.reshape(2 * N)
    slotr = slotr.reshape(2 * N)
    gate = gate.reshape(2 * N)

    xw = lax.bitcast_convert_type(
        x.astype(jnp.bfloat16).reshape(N, D // 2, 2), jnp.float32)
    bufw = _dispatch(xw, slotw, C, E)
    buf = lax.bitcast_convert_type(bufw, jnp.bfloat16).reshape(-1, D)
    y = _ffn(buf, W1.astype(jnp.bfloat16), b1, W2.astype(jnp.bfloat16), b2, C)
    out = _combine(y, slotr, gate, N, D)
    return out, aux[0, 0]


# speedup vs baseline: 2.8744x; 2.8744x over previous
import functools

import jax
import jax.numpy as jnp
from jax import lax
from jax.experimental import pallas as pl
from jax.experimental.pallas import tpu as pltpu
from jax.experimental.pallas import tpu_sc as plsc

K = 2
CF = 1.25
BAL_COEF = 0.01
Z_COEF = 0.001

try:
    _SC_INFO = plsc.get_sparse_core_info()
    _NC = _SC_INFO.num_cores
    _NS = _SC_INFO.num_subcores
    _LN = _SC_INFO.num_lanes
except Exception:
    _NC, _NS, _LN = 2, 16, 16
_NW = _NC * _NS


def _router_body(C, B, N, E, x_ref, wg_ref,
                 slotw_ref, slotr_ref, gate_ref, aux_ref,
                 runcnt_ref, imp_ref, kept_ref, z_ref):
    k = pl.program_id(0)
    b = pl.program_id(1)
    nb = pl.num_programs(1)

    @pl.when(jnp.logical_and(k == 0, b == 0))
    def _init():
        runcnt_ref[...] = jnp.zeros_like(runcnt_ref)
        imp_ref[...] = jnp.zeros_like(imp_ref)
        kept_ref[...] = jnp.zeros_like(kept_ref)
        z_ref[...] = jnp.zeros_like(z_ref)

    xb = x_ref[...]
    logits = jnp.dot(xb, wg_ref[...], preferred_element_type=jnp.float32)
    mx = jnp.max(logits, axis=1, keepdims=True)
    ex = jnp.exp(logits - mx)
    se = jnp.sum(ex, axis=1, keepdims=True)
    probs = ex / se
    lse = mx[:, 0] + jnp.log(se[:, 0])

    iota_e = lax.broadcasted_iota(jnp.int32, (B, E), 1)
    m1 = jnp.max(probs, axis=1, keepdims=True)
    i1 = jnp.min(jnp.where(probs == m1, iota_e, E), axis=1, keepdims=True)
    oh1 = iota_e == i1
    probs2 = jnp.where(oh1, -jnp.inf, probs)
    m2 = jnp.max(probs2, axis=1, keepdims=True)
    i2 = jnp.min(jnp.where(probs2 == m2, iota_e, E), axis=1, keepdims=True)

    is_k0 = k == 0
    ek = jnp.where(is_k0, i1, i2)
    ohk = iota_e == ek
    gk = jnp.where(is_k0, m1, m2)[:, 0] / (m1 + m2)[:, 0]

    ohf = ohk.astype(jnp.float32)
    r_i = lax.broadcasted_iota(jnp.int32, (B, B), 0)
    c_i = lax.broadcasted_iota(jnp.int32, (B, B), 1)
    tril = (r_i > c_i).astype(jnp.float32)
    prior = jnp.dot(tril, ohf, preferred_element_type=jnp.float32)
    pos = jnp.sum((runcnt_ref[...] + prior) * ohf, axis=1).astype(jnp.int32)
    runcnt_ref[...] = runcnt_ref[...] + jnp.sum(ohf, axis=0, keepdims=True)

    keep = pos < C
    e_flat = ek[:, 0]
    slotw_ref[...] = jnp.where(keep, e_flat * C + pos, E * C)[:, None]
    slotr_ref[...] = (e_flat * C + jnp.minimum(pos, C - 1))[:, None]
    gate_ref[...] = jnp.where(keep, gk, 0.0)[:, None]

    kept_ref[...] = kept_ref[...] + jnp.sum(
        jnp.where(keep[:, None], ohf, 0.0), axis=0, keepdims=True)

    @pl.when(is_k0)
    def _acc():
        imp_ref[...] = imp_ref[...] + jnp.sum(probs, axis=0, keepdims=True)
        z_ref[...] = z_ref[...] + jnp.sum(lse * lse).reshape(1, 1)

    @pl.when(jnp.logical_and(k == 1, b == nb - 1))
    def _final():
        imp = imp_ref[...] / N
        frac = kept_ref[...] / N
        bal = E * jnp.sum(frac * imp)
        z = z_ref[0, 0] / N
        aux_ref[...] = jnp.full((8, 128), BAL_COEF * bal + Z_COEF * z,
                                jnp.float32)


def _router(x, Wg, C, interpret=False):
    N, D = x.shape
    E = Wg.shape[1]
    B = 1024
    nb = N // B
    return pl.pallas_call(
        functools.partial(_router_body, C, B, N, E),
        grid=(2, nb),
        in_specs=[
            pl.BlockSpec((B, D), lambda k, b: (b, 0)),
            pl.BlockSpec((D, E), lambda k, b: (0, 0)),
        ],
        out_specs=[
            pl.BlockSpec((B, 1), lambda k, b, _nb=nb: (k * _nb + b, 0)),
            pl.BlockSpec((B, 1), lambda k, b, _nb=nb: (k * _nb + b, 0)),
            pl.BlockSpec((B, 1), lambda k, b, _nb=nb: (k * _nb + b, 0)),
            pl.BlockSpec((8, 128), lambda k, b: (0, 0)),
        ],
        out_shape=[
            jax.ShapeDtypeStruct((2 * N, 1), jnp.int32),
            jax.ShapeDtypeStruct((2 * N, 1), jnp.int32),
            jax.ShapeDtypeStruct((2 * N, 1), jnp.float32),
            jax.ShapeDtypeStruct((8, 128), jnp.float32),
        ],
        scratch_shapes=[
            pltpu.VMEM((1, E), jnp.float32),
            pltpu.VMEM((1, E), jnp.float32),
            pltpu.VMEM((1, E), jnp.float32),
            pltpu.VMEM((1, 1), jnp.float32),
        ],
        interpret=interpret,
    )(x, Wg)


def _dispatch(x, slotw, C, E):
    N, D = x.shape
    BD = 32
    steps = N // (_NW * BD)
    mesh = plsc.VectorSubcoreMesh(core_axis_name="c", subcore_axis_name="s")

    @functools.partial(
        pl.kernel,
        out_type=jax.ShapeDtypeStruct((E * C + 8, D), jnp.float32),
        mesh=mesh,
        compiler_params=pltpu.CompilerParams(needs_layout_passes=False),
        scratch_types=[
            pltpu.VMEM((BD, D), jnp.float32),
            pltpu.VMEM((BD, D), jnp.float32),
            pltpu.VMEM((BD,), jnp.int32),
            pltpu.VMEM((BD,), jnp.int32),
            pltpu.VMEM((BD,), jnp.int32),
            pltpu.VMEM((BD,), jnp.int32),
            pltpu.SemaphoreType.DMA,
            pltpu.SemaphoreType.DMA,
            pltpu.SemaphoreType.DMA,
            pltpu.SemaphoreType.DMA,
        ],
    )
    def _disp(x_hbm, slotw_hbm, buf_hbm, xv0, xv1, i0a, i0b, i1a, i1b,
              xsem0, xsem1, ssem0, ssem1):
        wid = lax.axis_index("s") * _NC + lax.axis_index("c")
        xv = (xv0, xv1)
        i0 = (i0a, i0b)
        i1 = (i1a, i1b)
        xsem = (xsem0, xsem1)
        ssem = (ssem0, ssem1)
        tok0 = wid * (steps * BD)

        def start_in(s, p):
            base = tok0 + s * BD
            return (
                pltpu.async_copy(x_hbm.at[pl.ds(base, BD)], xv[p], xsem[p]),
                pltpu.async_copy(slotw_hbm.at[pl.ds(base, BD)], i0[p],
                                 xsem[p]),
                pltpu.async_copy(slotw_hbm.at[pl.ds(N + base, BD)], i1[p],
                                 xsem[p]),
            )

        xcp = [start_in(0, 0), None]
        scat = [None, None]
        for s in range(steps):
            p = s & 1
            for cp in xcp[p]:
                cp.wait()
            scat[p] = (
                pltpu.async_copy(xv[p], buf_hbm.at[i0[p]], ssem[p]),
                pltpu.async_copy(xv[p], buf_hbm.at[i1[p]], ssem[p]),
            )
            if s >= 1:
                scat[1 - p][0].wait()
                scat[1 - p][1].wait()
            if s + 1 < steps:
                xcp[1 - p] = start_in(s + 1, 1 - p)
        scat[(steps - 1) & 1][0].wait()
        scat[(steps - 1) & 1][1].wait()

    return _disp(x, slotw)


def _ffn_body(buf_ref, w1_ref, b1_ref, w2_ref, b2_ref, y_ref):
    h = jnp.dot(buf_ref[...].astype(jnp.bfloat16), w1_ref[0],
                preferred_element_type=jnp.float32) + b1_ref[0]
    h = jax.nn.gelu(h).astype(jnp.bfloat16)
    y_ref[...] = jnp.dot(h, w2_ref[0],
                         preferred_element_type=jnp.float32) + b2_ref[0]


def _ffn(buf, W1, b1, W2, b2, C, interpret=False):
    E, D, DFF = W1.shape
    BC = 512
    nc = C // BC
    return pl.pallas_call(
        _ffn_body,
        grid=(E, nc),
        in_specs=[
            pl.BlockSpec((BC, D), lambda e, c, _nc=nc: (e * _nc + c, 0)),
            pl.BlockSpec((1, D, DFF), lambda e, c: (e, 0, 0)),
            pl.BlockSpec((1, 1, DFF), lambda e, c: (e, 0, 0)),
            pl.BlockSpec((1, DFF, D), lambda e, c: (e, 0, 0)),
            pl.BlockSpec((1, 1, D), lambda e, c: (e, 0, 0)),
        ],
        out_specs=pl.BlockSpec((BC, D), lambda e, c, _nc=nc: (e * _nc + c, 0)),
        out_shape=jax.ShapeDtypeStruct((E * C, D), jnp.float32),
        interpret=interpret,
    )(buf, W1, b1.reshape(E, 1, DFF), W2, b2.reshape(E, 1, D))


def _combine(y, slotr, gate, N, D):
    BC = 16
    tok_pw = N // _NW
    steps = tok_pw // BC
    mesh = plsc.VectorSubcoreMesh(core_axis_name="c", subcore_axis_name="s")

    @functools.partial(
        pl.kernel,
        out_type=jax.ShapeDtypeStruct((N, D), jnp.float32),
        mesh=mesh,
        compiler_params=pltpu.CompilerParams(needs_layout_passes=False),
        scratch_types=[
            pltpu.VMEM((BC, D), jnp.float32),
            pltpu.VMEM((BC, D), jnp.float32),
            pltpu.VMEM((BC, D), jnp.float32),
            pltpu.VMEM((BC, D), jnp.float32),
            pltpu.VMEM((BC, D), jnp.float32),
            pltpu.VMEM((BC, D), jnp.float32),
            pltpu.VMEM((tok_pw,), jnp.int32),
            pltpu.VMEM((tok_pw,), jnp.int32),
            pltpu.VMEM((tok_pw,), jnp.float32),
            pltpu.VMEM((tok_pw,), jnp.float32),
            pltpu.SemaphoreType.DMA,
            pltpu.SemaphoreType.DMA,
            pltpu.SemaphoreType.DMA,
            pltpu.SemaphoreType.DMA,
        ],
    )
    def _comb(y_hbm, slotr_hbm, gate_hbm, out_hbm,
              y0a, y0b, y1a, y1b, ova, ovb, i0all, i1all, g0all, g1all,
              gsem0, gsem1, osem0, osem1):
        wid = lax.axis_index("s") * _NC + lax.axis_index("c")
        tok0 = wid * tok_pw
        y0 = (y0a, y0b)
        y1 = (y1a, y1b)
        ov = (ova, ovb)
        gsem = (gsem0, gsem1)
        osem = (osem0, osem1)

        pltpu.sync_copy(slotr_hbm.at[pl.ds(tok0, tok_pw)], i0all)
        pltpu.sync_copy(slotr_hbm.at[pl.ds(N + tok0, tok_pw)], i1all)
        pltpu.sync_copy(gate_hbm.at[pl.ds(tok0, tok_pw)], g0all)
        pltpu.sync_copy(gate_hbm.at[pl.ds(N + tok0, tok_pw)], g1all)

        def start_gathers(s, p):
            pltpu.async_copy(
                y_hbm.at[i0all.at[pl.ds(s * BC, BC)]], y0[p], gsem[p])
            pltpu.async_copy(
                y_hbm.at[i1all.at[pl.ds(s * BC, BC)]], y1[p], gsem[p])

        start_gathers(0, 0)
        start_gathers(1, 1)

        def outer(i, carry):
            for p in (0, 1):
                s = 2 * i + p
                pltpu.make_async_copy(
                    y_hbm.at[pl.ds(0, BC)], y0[p], gsem[p]).wait()
                pltpu.make_async_copy(
                    y_hbm.at[pl.ds(0, BC)], y1[p], gsem[p]).wait()

                @pl.when(s >= 2)
                def _wait_out(p=p):
                    pltpu.make_async_copy(
                        ov[p], out_hbm.at[pl.ds(tok0, BC)], osem[p]).wait()

                def tbody(t, c, p=p, s=s):
                    tt = jnp.zeros((_LN,), jnp.int32) + (s * BC + t)
                    g0 = plsc.load_gather(g0all, [tt])
                    g1 = plsc.load_gather(g1all, [tt])
                    for cc in range(D // _LN):
                        sl = pl.ds(cc * _LN, _LN)
                        ov[p][t, sl] = g0 * y0[p][t, sl] + g1 * y1[p][t, sl]
                    return c

                lax.fori_loop(0, BC, tbody, 0)
                pltpu.async_copy(
                    ov[p], out_hbm.at[pl.ds(tok0 + s * BC, BC)], osem[p])

                @pl.when(s + 2 < steps)
                def _next(p=p, s=s):
                    start_gathers(s + 2, p)
            return carry

        lax.fori_loop(0, steps // 2, outer, 0)
        for p in (0, 1):
            pltpu.make_async_copy(
                ov[p], out_hbm.at[pl.ds(tok0, BC)], osem[p]).wait()

    return _comb(y, slotr, gate)


def kernel(x, Wg, W1, b1, W2, b2):
    N, D = x.shape
    E = Wg.shape[1]
    C = int(CF * N * K / E)

    slotw, slotr, gate, aux = _router(x, Wg, C)
    slotw = slotw.reshape(2 * N)
    slotr = slotr.reshape(2 * N)
    gate = gate.reshape(2 * N)

    buf = _dispatch(x, slotw, C, E)
    y = _ffn(buf, W1.astype(jnp.bfloat16), b1, W2.astype(jnp.bfloat16), b2, C)
    out = _combine(y, slotr, gate, N, D)
    return out, aux[0, 0]


# speedup vs baseline: 3.0224x; 1.0515x over previous
import functools

import jax
import jax.numpy as jnp
from jax import lax
from jax.experimental import pallas as pl
from jax.experimental.pallas import tpu as pltpu
from jax.experimental.pallas import tpu_sc as plsc

K = 2
CF = 1.25
BAL_COEF = 0.01
Z_COEF = 0.001

try:
    _SC_INFO = plsc.get_sparse_core_info()
    _NC = _SC_INFO.num_cores
    _NS = _SC_INFO.num_subcores
    _LN = _SC_INFO.num_lanes
except Exception:
    _NC, _NS, _LN = 2, 16, 16
_NW = _NC * _NS


def _router_body(C, B, N, E, x_ref, wg_ref,
                 slotw_ref, slotr_ref, gate_ref, aux_ref,
                 runcnt_ref, imp_ref, kept_ref, z_ref):
    k = pl.program_id(0)
    b = pl.program_id(1)
    nb = pl.num_programs(1)

    @pl.when(jnp.logical_and(k == 0, b == 0))
    def _init():
        runcnt_ref[...] = jnp.zeros_like(runcnt_ref)
        imp_ref[...] = jnp.zeros_like(imp_ref)
        kept_ref[...] = jnp.zeros_like(kept_ref)
        z_ref[...] = jnp.zeros_like(z_ref)

    xb = x_ref[...]
    logits = jnp.dot(xb, wg_ref[...], preferred_element_type=jnp.float32)
    mx = jnp.max(logits, axis=1, keepdims=True)
    ex = jnp.exp(logits - mx)
    se = jnp.sum(ex, axis=1, keepdims=True)
    probs = ex / se
    lse = mx[:, 0] + jnp.log(se[:, 0])

    iota_e = lax.broadcasted_iota(jnp.int32, (B, E), 1)
    m1 = jnp.max(probs, axis=1, keepdims=True)
    i1 = jnp.min(jnp.where(probs == m1, iota_e, E), axis=1, keepdims=True)
    oh1 = iota_e == i1
    probs2 = jnp.where(oh1, -jnp.inf, probs)
    m2 = jnp.max(probs2, axis=1, keepdims=True)
    i2 = jnp.min(jnp.where(probs2 == m2, iota_e, E), axis=1, keepdims=True)

    is_k0 = k == 0
    ek = jnp.where(is_k0, i1, i2)
    ohk = iota_e == ek
    gk = jnp.where(is_k0, m1, m2)[:, 0] / (m1 + m2)[:, 0]

    ohf = ohk.astype(jnp.float32)
    r_i = lax.broadcasted_iota(jnp.int32, (B, B), 0)
    c_i = lax.broadcasted_iota(jnp.int32, (B, B), 1)
    tril = (r_i > c_i).astype(jnp.float32)
    prior = jnp.dot(tril, ohf, preferred_element_type=jnp.float32)
    pos = jnp.sum((runcnt_ref[...] + prior) * ohf, axis=1).astype(jnp.int32)
    runcnt_ref[...] = runcnt_ref[...] + jnp.sum(ohf, axis=0, keepdims=True)

    keep = pos < C
    e_flat = ek[:, 0]
    slotw_ref[...] = jnp.where(keep, e_flat * C + pos, E * C)[:, None]
    slotr_ref[...] = (e_flat * C + jnp.minimum(pos, C - 1))[:, None]
    gate_ref[...] = jnp.where(keep, gk, 0.0)[:, None]

    kept_ref[...] = kept_ref[...] + jnp.sum(
        jnp.where(keep[:, None], ohf, 0.0), axis=0, keepdims=True)

    @pl.when(is_k0)
    def _acc():
        imp_ref[...] = imp_ref[...] + jnp.sum(probs, axis=0, keepdims=True)
        z_ref[...] = z_ref[...] + jnp.sum(lse * lse).reshape(1, 1)

    @pl.when(jnp.logical_and(k == 1, b == nb - 1))
    def _final():
        imp = imp_ref[...] / N
        frac = kept_ref[...] / N
        bal = E * jnp.sum(frac * imp)
        z = z_ref[0, 0] / N
        aux_ref[...] = jnp.full((8, 128), BAL_COEF * bal + Z_COEF * z,
                                jnp.float32)


def _router(x, Wg, C, interpret=False):
    N, D = x.shape
    E = Wg.shape[1]
    B = 1024
    nb = N // B
    return pl.pallas_call(
        functools.partial(_router_body, C, B, N, E),
        grid=(2, nb),
        in_specs=[
            pl.BlockSpec((B, D), lambda k, b: (b, 0)),
            pl.BlockSpec((D, E), lambda k, b: (0, 0)),
        ],
        out_specs=[
            pl.BlockSpec((B, 1), lambda k, b, _nb=nb: (k * _nb + b, 0)),
            pl.BlockSpec((B, 1), lambda k, b, _nb=nb: (k * _nb + b, 0)),
            pl.BlockSpec((B, 1), lambda k, b, _nb=nb: (k * _nb + b, 0)),
            pl.BlockSpec((8, 128), lambda k, b: (0, 0)),
        ],
        out_shape=[
            jax.ShapeDtypeStruct((2 * N, 1), jnp.int32),
            jax.ShapeDtypeStruct((2 * N, 1), jnp.int32),
            jax.ShapeDtypeStruct((2 * N, 1), jnp.float32),
            jax.ShapeDtypeStruct((8, 128), jnp.float32),
        ],
        scratch_shapes=[
            pltpu.VMEM((1, E), jnp.float32),
            pltpu.VMEM((1, E), jnp.float32),
            pltpu.VMEM((1, E), jnp.float32),
            pltpu.VMEM((1, 1), jnp.float32),
        ],
        interpret=interpret,
    )(x, Wg)


def _dispatch(x, slotw, C, E):
    N, D = x.shape
    BD = 32
    steps = N // (_NW * BD)
    mesh = plsc.VectorSubcoreMesh(core_axis_name="c", subcore_axis_name="s")

    @functools.partial(
        pl.kernel,
        out_type=jax.ShapeDtypeStruct((E * C + 8, D), jnp.float32),
        mesh=mesh,
        compiler_params=pltpu.CompilerParams(needs_layout_passes=False),
        scratch_types=[
            pltpu.VMEM((BD, D), jnp.float32),
            pltpu.VMEM((BD, D), jnp.float32),
            pltpu.VMEM((BD,), jnp.int32),
            pltpu.VMEM((BD,), jnp.int32),
            pltpu.VMEM((BD,), jnp.int32),
            pltpu.VMEM((BD,), jnp.int32),
            pltpu.SemaphoreType.DMA,
            pltpu.SemaphoreType.DMA,
            pltpu.SemaphoreType.DMA,
            pltpu.SemaphoreType.DMA,
        ],
    )
    def _disp(x_hbm, slotw_hbm, buf_hbm, xv0, xv1, i0a, i0b, i1a, i1b,
              xsem0, xsem1, ssem0, ssem1):
        wid = lax.axis_index("s") * _NC + lax.axis_index("c")
        xv = (xv0, xv1)
        i0 = (i0a, i0b)
        i1 = (i1a, i1b)
        xsem = (xsem0, xsem1)
        ssem = (ssem0, ssem1)
        tok0 = wid * (steps * BD)

        def start_in(s, p):
            base = tok0 + s * BD
            return (
                pltpu.async_copy(x_hbm.at[pl.ds(base, BD)], xv[p], xsem[p]),
                pltpu.async_copy(slotw_hbm.at[pl.ds(base, BD)], i0[p],
                                 xsem[p]),
                pltpu.async_copy(slotw_hbm.at[pl.ds(N + base, BD)], i1[p],
                                 xsem[p]),
            )

        xcp = [start_in(0, 0), None]
        scat = [None, None]
        for s in range(steps):
            p = s & 1
            for cp in xcp[p]:
                cp.wait()
            scat[p] = (
                pltpu.async_copy(xv[p], buf_hbm.at[i0[p]], ssem[p]),
                pltpu.async_copy(xv[p], buf_hbm.at[i1[p]], ssem[p]),
            )
            if s >= 1:
                scat[1 - p][0].wait()
                scat[1 - p][1].wait()
            if s + 1 < steps:
                xcp[1 - p] = start_in(s + 1, 1 - p)
        scat[(steps - 1) & 1][0].wait()
        scat[(steps - 1) & 1][1].wait()

    return _disp(x, slotw)


def _ffn_body(buf_ref, w1_ref, b1_ref, w2_ref, b2_ref, y_ref):
    h = jnp.dot(buf_ref[...], w1_ref[0],
                preferred_element_type=jnp.float32) + b1_ref[0]
    h = jax.nn.gelu(h)
    y_ref[...] = jnp.dot(h, w2_ref[0],
                         preferred_element_type=jnp.float32) + b2_ref[0]


def _ffn(buf, W1, b1, W2, b2, C, interpret=False):
    E, D, DFF = W1.shape
    BC = 512
    nc = C // BC
    return pl.pallas_call(
        _ffn_body,
        grid=(E, nc),
        in_specs=[
            pl.BlockSpec((BC, D), lambda e, c, _nc=nc: (e * _nc + c, 0)),
            pl.BlockSpec((1, D, DFF), lambda e, c: (e, 0, 0)),
            pl.BlockSpec((1, 1, DFF), lambda e, c: (e, 0, 0)),
            pl.BlockSpec((1, DFF, D), lambda e, c: (e, 0, 0)),
            pl.BlockSpec((1, 1, D), lambda e, c: (e, 0, 0)),
        ],
        out_specs=pl.BlockSpec((BC, D), lambda e, c, _nc=nc: (e * _nc + c, 0)),
        out_shape=jax.ShapeDtypeStruct((E * C, D), jnp.float32),
        interpret=interpret,
    )(buf, W1, b1.reshape(E, 1, DFF), W2, b2.reshape(E, 1, D))


def _combine(y, slotr, gate, N, D):
    BC = 16
    tok_pw = N // _NW
    steps = tok_pw // BC
    mesh = plsc.VectorSubcoreMesh(core_axis_name="c", subcore_axis_name="s")

    @functools.partial(
        pl.kernel,
        out_type=jax.ShapeDtypeStruct((N, D), jnp.float32),
        mesh=mesh,
        compiler_params=pltpu.CompilerParams(needs_layout_passes=False),
        scratch_types=[
            pltpu.VMEM((BC, D), jnp.float32),
            pltpu.VMEM((BC, D), jnp.float32),
            pltpu.VMEM((BC, D), jnp.float32),
            pltpu.VMEM((BC, D), jnp.float32),
            pltpu.VMEM((BC, D), jnp.float32),
            pltpu.VMEM((BC, D), jnp.float32),
            pltpu.VMEM((tok_pw,), jnp.int32),
            pltpu.VMEM((tok_pw,), jnp.int32),
            pltpu.VMEM((tok_pw,), jnp.float32),
            pltpu.VMEM((tok_pw,), jnp.float32),
            pltpu.SemaphoreType.DMA,
            pltpu.SemaphoreType.DMA,
            pltpu.SemaphoreType.DMA,
            pltpu.SemaphoreType.DMA,
        ],
    )
    def _comb(y_hbm, slotr_hbm, gate_hbm, out_hbm,
              y0a, y0b, y1a, y1b, ova, ovb, i0all, i1all, g0all, g1all,
              gsem0, gsem1, osem0, osem1):
        wid = lax.axis_index("s") * _NC + lax.axis_index("c")
        tok0 = wid * tok_pw
        y0 = (y0a, y0b)
        y1 = (y1a, y1b)
        ov = (ova, ovb)
        gsem = (gsem0, gsem1)
        osem = (osem0, osem1)

        pltpu.sync_copy(slotr_hbm.at[pl.ds(tok0, tok_pw)], i0all)
        pltpu.sync_copy(slotr_hbm.at[pl.ds(N + tok0, tok_pw)], i1all)
        pltpu.sync_copy(gate_hbm.at[pl.ds(tok0, tok_pw)], g0all)
        pltpu.sync_copy(gate_hbm.at[pl.ds(N + tok0, tok_pw)], g1all)

        def start_gathers(s, p):
            pltpu.async_copy(
                y_hbm.at[i0all.at[pl.ds(s * BC, BC)]], y0[p], gsem[p])
            pltpu.async_copy(
                y_hbm.at[i1all.at[pl.ds(s * BC, BC)]], y1[p], gsem[p])

        start_gathers(0, 0)
        start_gathers(1, 1)

        def outer(i, carry):
            for p in (0, 1):
                s = 2 * i + p
                pltpu.make_async_copy(
                    y_hbm.at[pl.ds(0, BC)], y0[p], gsem[p]).wait()
                pltpu.make_async_copy(
                    y_hbm.at[pl.ds(0, BC)], y1[p], gsem[p]).wait()

                @pl.when(s >= 2)
                def _wait_out(p=p):
                    pltpu.make_async_copy(
                        ov[p], out_hbm.at[pl.ds(tok0, BC)], osem[p]).wait()

                def tbody(t, c, p=p, s=s):
                    tt = jnp.zeros((_LN,), jnp.int32) + (s * BC + t)
                    g0 = plsc.load_gather(g0all, [tt])
                    g1 = plsc.load_gather(g1all, [tt])
                    for cc in range(D // _LN):
                        sl = pl.ds(cc * _LN, _LN)
                        ov[p][t, sl] = g0 * y0[p][t, sl] + g1 * y1[p][t, sl]
                    return c

                lax.fori_loop(0, BC, tbody, 0)
                pltpu.async_copy(
                    ov[p], out_hbm.at[pl.ds(tok0 + s * BC, BC)], osem[p])

                @pl.when(s + 2 < steps)
                def _next(p=p, s=s):
                    start_gathers(s + 2, p)
            return carry

        lax.fori_loop(0, steps // 2, outer, 0)
        for p in (0, 1):
            pltpu.make_async_copy(
                ov[p], out_hbm.at[pl.ds(tok0, BC)], osem[p]).wait()

    return _comb(y, slotr, gate)


def kernel(x, Wg, W1, b1, W2, b2):
    N, D = x.shape
    E = Wg.shape[1]
    C = int(CF * N * K / E)

    slotw, slotr, gate, aux = _router(x, Wg, C)
    slotw = slotw.reshape(2 * N)
    slotr = slotr.reshape(2 * N)
    gate = gate.reshape(2 * N)

    buf = _dispatch(x, slotw, C, E)
    y = _ffn(buf, W1, b1, W2, b2, C)
    out = _combine(y, slotr, gate, N, D)
    return out, aux[0, 0]
